# baseline (speedup 1.0000x reference)
import functools

import jax
import jax.numpy as jnp
from jax import lax
from jax.experimental import pallas as pl
from jax.experimental.pallas import tpu as pltpu
from jax.experimental.pallas import tpu_sc as plsc

N_NODES = 10000
NUM_CORES = 2
NUM_SUBCORES = 16
NUM_TILES = NUM_CORES * NUM_SUBCORES
CHUNK = 128
ROWS_PER_TILE = 640
H_PAD = ROWS_PER_TILE * NUM_SUBCORES
BLK = 1024
_LN_EPS = 1e-5


NBUF = 2
IDXW = 16


def _sc_degree(srcs, ones128, zeros128):
    cpt = srcs.shape[1]
    mesh = plsc.VectorSubcoreMesh(core_axis_name="c", subcore_axis_name="s")

    @functools.partial(
        pl.kernel,
        out_type=jax.ShapeDtypeStruct((NUM_CORES, H_PAD, 128), jnp.float32),
        mesh=mesh,
        scratch_types=[
            pltpu.VMEM((CHUNK,), jnp.int32),
            pltpu.VMEM((CHUNK, 128), jnp.float32),
            pltpu.VMEM_SHARED((H_PAD, 128), jnp.float32),
        ],
    )
    def deg_kernel(src_hbm, ones_hbm, z_hbm, out_hbm, sbuf, ones_v, acc):
        cid = lax.axis_index("c")
        sid = lax.axis_index("s")
        wid = sid * NUM_CORES + cid
        pltpu.sync_copy(ones_hbm, ones_v)
        for k in range(ROWS_PER_TILE // CHUNK):
            pltpu.sync_copy(
                z_hbm, acc.at[pl.ds(sid * ROWS_PER_TILE + k * CHUNK, CHUNK)])
        plsc.subcore_barrier()

        @pl.loop(0, cpt)
        def _(j):
            pltpu.sync_copy(src_hbm.at[wid, j], sbuf)
            pltpu.sync_copy(ones_v, acc.at[sbuf], add=True)

        plsc.subcore_barrier()
        pltpu.sync_copy(
            acc.at[pl.ds(sid * ROWS_PER_TILE, ROWS_PER_TILE)],
            out_hbm.at[cid, pl.ds(sid * ROWS_PER_TILE, ROWS_PER_TILE)])

    return deg_kernel(srcs, ones128, zeros128)


def _sc_scatter(u, srcs, dsts, zeros128):
    cpt = srcs.shape[1]
    assert cpt % NBUF == 0
    mesh = plsc.VectorSubcoreMesh(core_axis_name="c", subcore_axis_name="s")

    scratch = ([pltpu.VMEM((CHUNK,), jnp.int32)] * 2
               + [pltpu.VMEM((CHUNK, 128), jnp.float32)]
               + [pltpu.VMEM_SHARED((H_PAD, 128), jnp.float32)])

    @functools.partial(
        pl.kernel,
        out_type=jax.ShapeDtypeStruct((NUM_CORES, H_PAD, 128), jnp.float32),
        mesh=mesh,
        scratch_types=scratch,
    )
    def scat_kernel(u_hbm, src_hbm, dst_hbm, z_hbm, out_hbm,
                    sbuf, dbuf, rows, acc):
        cid = lax.axis_index("c")
        sid = lax.axis_index("s")
        wid = sid * NUM_CORES + cid
        pltpu.sync_copy(z_hbm, rows)
        for k in range(ROWS_PER_TILE // CHUNK):
            pltpu.sync_copy(
                rows, acc.at[pl.ds(sid * ROWS_PER_TILE + k * CHUNK, CHUNK)])
        plsc.subcore_barrier()

        @pl.loop(0, cpt)
        def _(j):
            pltpu.sync_copy(src_hbm.at[wid, j], sbuf)
            pltpu.sync_copy(dst_hbm.at[wid, j], dbuf)
            pltpu.sync_copy(u_hbm.at[sbuf], rows)
            pltpu.sync_copy(rows, acc.at[dbuf], add=True)

        plsc.subcore_barrier()
        pltpu.sync_copy(
            acc.at[pl.ds(sid * ROWS_PER_TILE, ROWS_PER_TILE)],
            out_hbm.at[cid, pl.ds(sid * ROWS_PER_TILE, ROWS_PER_TILE)])

    return scat_kernel(u, srcs, dsts, zeros128)


def _tc_prep(deg_p, x_pad):
    def body(dp_ref, x_ref, dis_ref, u_ref):
        deg = dp_ref[0, :, 0:1] + dp_ref[1, :, 0:1]
        dis = jnp.where(deg > 0, lax.rsqrt(deg), 0.0)
        dis_ref[...] = dis
        u_ref[...] = dis * x_ref[...]

    return pl.pallas_call(
        body,
        grid=(H_PAD // BLK,),
        in_specs=[pl.BlockSpec((2, BLK, 128), lambda i: (0, i, 0)),
                  pl.BlockSpec((BLK, 128), lambda i: (i, 0))],
        out_specs=[pl.BlockSpec((BLK, 1), lambda i: (i, 0)),
                   pl.BlockSpec((BLK, 128), lambda i: (i, 0))],
        out_shape=[jax.ShapeDtypeStruct((H_PAD, 1), jnp.float32),
                   jax.ShapeDtypeStruct((H_PAD, 128), jnp.float32)],
    )(deg_p, x_pad)


def _tc_matmul(h, w, b=None, dis=None):
    din, dout = w.shape
    n_in = 2 + (b is not None) + (dis is not None)

    def body(*refs):
        h_ref, w_ref = refs[0], refs[1]
        rest = list(refs[2:-1])
        o_ref = refs[-1]
        m = jnp.dot(h_ref[...], w_ref[...], preferred_element_type=jnp.float32,
                    precision=lax.Precision.HIGHEST)
        if b is not None:
            m = m + rest.pop(0)[...]
        if dis is not None:
            m = rest.pop(0)[...] * m
        o_ref[...] = m

    in_specs = [pl.BlockSpec((BLK, din), lambda i: (i, 0)),
                pl.BlockSpec((din, dout), lambda i: (0, 0))]
    args = [h, w]
    if b is not None:
        in_specs.append(pl.BlockSpec((1, dout), lambda i: (0, 0)))
        args.append(b.reshape(1, dout))
    if dis is not None:
        in_specs.append(pl.BlockSpec((BLK, 1), lambda i: (i, 0)))
        args.append(dis)
    del n_in
    return pl.pallas_call(
        body,
        grid=(H_PAD // BLK,),
        in_specs=in_specs,
        out_specs=pl.BlockSpec((BLK, dout), lambda i: (i, 0)),
        out_shape=jax.ShapeDtypeStruct((H_PAD, dout), jnp.float32),
    )(*args)


def _tc_combine(m, parts, dis, wb, ln_g, ln_b, do_ln, split_u):
    dout = m.shape[1]
    nparts = len(parts)

    def body(*refs):
        i = 0
        m_ref = refs[i]; i += 1
        p_refs = refs[i:i + nparts]; i += nparts
        dis_ref = refs[i]; i += 1
        wb_refs = refs[i:i + (nparts if wb is not None else 0)]
        i += len(wb_refs)
        if do_ln:
            lg_ref = refs[i]; i += 1
            lb_ref = refs[i]; i += 1
        h_ref = refs[i]; i += 1
        u_refs = refs[i:]

        dis_v = dis_ref[...]
        p = m_ref[...]
        for k in range(nparts):
            g = -dis_v * (p_refs[k][0] + p_refs[k][1])
            if wb is not None:
                p = p + jnp.dot(g, wb_refs[k][...],
                                preferred_element_type=jnp.float32,
                                precision=lax.Precision.HIGHEST)
            else:
                p = p + g
        if do_ln:
            mu = jnp.mean(p, axis=-1, keepdims=True)
            d = p - mu
            var = jnp.mean(d * d, axis=-1, keepdims=True)
            p = d * lax.rsqrt(var + _LN_EPS) * lg_ref[...] + lb_ref[...]
        pos = p > 0
        h = jnp.where(pos, p, jnp.exp(jnp.where(pos, 0.0, p)) - 1.0)
        h_ref[...] = h
        if split_u:
            u = dis_v * h
            u_refs[0][...] = u[:, :128]
            u_refs[1][...] = u[:, 128:]

    in_specs = [pl.BlockSpec((BLK, dout), lambda i: (i, 0))]
    args = [m]
    for pt in parts:
        in_specs.append(pl.BlockSpec((2, BLK, 128), lambda i: (0, i, 0)))
        args.append(pt)
    in_specs.append(pl.BlockSpec((BLK, 1), lambda i: (i, 0)))
    args.append(dis)
    if wb is not None:
        for k in range(nparts):
            in_specs.append(pl.BlockSpec((128, dout), lambda i: (0, 0)))
            args.append(wb[k * 128:(k + 1) * 128])
    if do_ln:
        in_specs.append(pl.BlockSpec((1, dout), lambda i: (0, 0)))
        args.append(ln_g.reshape(1, dout))
        in_specs.append(pl.BlockSpec((1, dout), lambda i: (0, 0)))
        args.append(ln_b.reshape(1, dout))

    out_specs = [pl.BlockSpec((BLK, dout), lambda i: (i, 0))]
    out_shape = [jax.ShapeDtypeStruct((H_PAD, dout), jnp.float32)]
    if split_u:
        for _ in range(2):
            out_specs.append(pl.BlockSpec((BLK, 128), lambda i: (i, 0)))
            out_shape.append(jax.ShapeDtypeStruct((H_PAD, 128), jnp.float32))

    out = pl.pallas_call(
        body,
        grid=(H_PAD // BLK,),
        in_specs=in_specs,
        out_specs=out_specs,
        out_shape=out_shape,
    )(*args)
    return out if split_u else (out[0],)


def kernel(x, edge_index, params):
    x = x.astype(jnp.float32)
    src = edge_index[0].astype(jnp.int32)
    dst = edge_index[1].astype(jnp.int32)
    e = src.shape[0]
    cpt = -(-e // (NUM_TILES * CHUNK))
    cpt = -(-cpt // IDXW) * IDXW
    e_pad = NUM_TILES * CHUNK * cpt
    pad = e_pad - e
    pad_idx = jnp.full((pad,), N_NODES, jnp.int32)
    srcs = jnp.concatenate([src, pad_idx]).reshape(NUM_TILES, cpt, CHUNK)
    dsts = jnp.concatenate([dst, pad_idx]).reshape(NUM_TILES, cpt, CHUNK)
    x_pad = jnp.pad(x, ((0, H_PAD - x.shape[0]), (0, 0)))

    zeros128 = jnp.zeros((CHUNK, 128), jnp.float32)
    ones128 = jnp.ones((CHUNK, 128), jnp.float32)

    deg_p = _sc_degree(srcs, ones128, zeros128)
    dis, u0 = _tc_prep(deg_p, x_pad)

    layers = params["layers"]
    lns = params["ln"]
    num_layers = len(layers)

    h = x_pad
    u_halves = [u0]
    for li in range(num_layers):
        p0, p1 = layers[li][0], layers[li][1]
        s0 = jax.nn.sigmoid(p0["sw"])[0]
        s1 = jax.nn.sigmoid(p1["sw"])[0]
        wa = (p0["Ws"][0] * s0 + p1["Ws"][0] * s1) * 0.5
        wb = p1["Ws"][1] * (s1 * 0.5)
        bb = (p0["b"] * s0 + p1["b"] * s1) * 0.5
        last = li == num_layers - 1

        m = _tc_matmul(h, wa, b=bb)
        if last:
            u3 = _tc_matmul(h, wb, dis=dis)
            parts = [_sc_scatter(u3, srcs, dsts, zeros128)]
            (h,) = _tc_combine(m, parts, dis, None, None, None,
                               do_ln=False, split_u=False)
        else:
            parts = [_sc_scatter(u, srcs, dsts, zeros128) for u in u_halves]
            res = _tc_combine(m, parts, dis, wb, lns[li]["g"], lns[li]["b"],
                              do_ln=True, split_u=(li < num_layers - 2))
            if li < num_layers - 2:
                h, ua, ub = res
                u_halves = [ua, ub]
            else:
                (h,) = res
    return h[:N_NODES]


# speedup vs baseline: 2.3316x; 2.3316x over previous
import functools

import jax
import jax.numpy as jnp
from jax import lax
from jax.experimental import pallas as pl
from jax.experimental.pallas import tpu as pltpu
from jax.experimental.pallas import tpu_sc as plsc

N_NODES = 10000
NUM_CORES = 2
NUM_SUBCORES = 16
NUM_TILES = NUM_CORES * NUM_SUBCORES
CHUNK = 128
ROWS_PER_TILE = 640
H_PAD = ROWS_PER_TILE * NUM_SUBCORES
BLK = 1024
_LN_EPS = 1e-5


NBUF = 2
IDXW = 16


def _sc_degree(srcs, ones128, zeros128):
    cpt = srcs.shape[1]
    mesh = plsc.VectorSubcoreMesh(core_axis_name="c", subcore_axis_name="s")

    @functools.partial(
        pl.kernel,
        out_type=jax.ShapeDtypeStruct((NUM_CORES, H_PAD, 128), jnp.float32),
        mesh=mesh,
        scratch_types=[
            pltpu.VMEM((CHUNK,), jnp.int32),
            pltpu.VMEM((CHUNK, 128), jnp.float32),
            pltpu.VMEM_SHARED((H_PAD, 128), jnp.float32),
        ],
    )
    def deg_kernel(src_hbm, ones_hbm, z_hbm, out_hbm, sbuf, ones_v, acc):
        cid = lax.axis_index("c")
        sid = lax.axis_index("s")
        wid = sid * NUM_CORES + cid
        pltpu.sync_copy(ones_hbm, ones_v)
        for k in range(ROWS_PER_TILE // CHUNK):
            pltpu.sync_copy(
                z_hbm, acc.at[pl.ds(sid * ROWS_PER_TILE + k * CHUNK, CHUNK)])
        plsc.subcore_barrier()

        @pl.loop(0, cpt)
        def _(j):
            pltpu.sync_copy(src_hbm.at[wid, j], sbuf)
            pltpu.sync_copy(ones_v, acc.at[sbuf], add=True)

        plsc.subcore_barrier()
        pltpu.sync_copy(
            acc.at[pl.ds(sid * ROWS_PER_TILE, ROWS_PER_TILE)],
            out_hbm.at[cid, pl.ds(sid * ROWS_PER_TILE, ROWS_PER_TILE)])

    return deg_kernel(srcs, ones128, zeros128)


def _sc_scatter(u, srcs, dsts, zeros128):
    cpt = srcs.shape[1]
    mesh = plsc.VectorSubcoreMesh(core_axis_name="c", subcore_axis_name="s")

    scratch = ([pltpu.VMEM((CHUNK,), jnp.int32)] * 2
               + [pltpu.VMEM((CHUNK, 128), jnp.float32)]
               + [pltpu.VMEM_SHARED((H_PAD, 128), jnp.float32)])

    @functools.partial(
        pl.kernel,
        out_type=jax.ShapeDtypeStruct((NUM_CORES, H_PAD, 128), jnp.float32),
        mesh=mesh,
        scratch_types=scratch,
    )
    def scat_kernel(u_hbm, src_hbm, dst_hbm, z_hbm, out_hbm,
                    sbuf, dbuf, rows, acc):
        cid = lax.axis_index("c")
        sid = lax.axis_index("s")
        wid = sid * NUM_CORES + cid
        pltpu.sync_copy(z_hbm, rows)
        for k in range(ROWS_PER_TILE // CHUNK):
            pltpu.sync_copy(
                rows, acc.at[pl.ds(sid * ROWS_PER_TILE + k * CHUNK, CHUNK)])
        plsc.subcore_barrier()

        @pl.loop(0, cpt)
        def _(j):
            pltpu.sync_copy(src_hbm.at[wid, j], sbuf)
            pltpu.sync_copy(dst_hbm.at[wid, j], dbuf)
            pltpu.sync_copy(u_hbm.at[sbuf], rows)
            pltpu.sync_copy(rows, acc.at[dbuf], add=True)

        plsc.subcore_barrier()
        pltpu.sync_copy(
            acc.at[pl.ds(sid * ROWS_PER_TILE, ROWS_PER_TILE)],
            out_hbm.at[cid, pl.ds(sid * ROWS_PER_TILE, ROWS_PER_TILE)])

    return scat_kernel(u, srcs, dsts, zeros128)


def _tc_prep(deg_p, x_pad):
    def body(dp_ref, x_ref, dis_ref, u_ref):
        deg = dp_ref[0, :, 0:1] + dp_ref[1, :, 0:1]
        dis = jnp.where(deg > 0, lax.rsqrt(deg), 0.0)
        dis_ref[...] = dis
        u_ref[...] = dis * x_ref[...]

    return pl.pallas_call(
        body,
        grid=(H_PAD // BLK,),
        in_specs=[pl.BlockSpec((2, BLK, 128), lambda i: (0, i, 0)),
                  pl.BlockSpec((BLK, 128), lambda i: (i, 0))],
        out_specs=[pl.BlockSpec((BLK, 1), lambda i: (i, 0)),
                   pl.BlockSpec((BLK, 128), lambda i: (i, 0))],
        out_shape=[jax.ShapeDtypeStruct((H_PAD, 1), jnp.float32),
                   jax.ShapeDtypeStruct((H_PAD, 128), jnp.float32)],
    )(deg_p, x_pad)


def _tc_matmul(h, w, b=None, dis=None):
    din, dout = w.shape
    n_in = 2 + (b is not None) + (dis is not None)

    def body(*refs):
        h_ref, w_ref = refs[0], refs[1]
        rest = list(refs[2:-1])
        o_ref = refs[-1]
        m = jnp.dot(h_ref[...], w_ref[...], preferred_element_type=jnp.float32,
                    precision=lax.Precision.HIGHEST)
        if b is not None:
            m = m + rest.pop(0)[...]
        if dis is not None:
            m = rest.pop(0)[...] * m
        o_ref[...] = m

    in_specs = [pl.BlockSpec((BLK, din), lambda i: (i, 0)),
                pl.BlockSpec((din, dout), lambda i: (0, 0))]
    args = [h, w]
    if b is not None:
        in_specs.append(pl.BlockSpec((1, dout), lambda i: (0, 0)))
        args.append(b.reshape(1, dout))
    if dis is not None:
        in_specs.append(pl.BlockSpec((BLK, 1), lambda i: (i, 0)))
        args.append(dis)
    del n_in
    return pl.pallas_call(
        body,
        grid=(H_PAD // BLK,),
        in_specs=in_specs,
        out_specs=pl.BlockSpec((BLK, dout), lambda i: (i, 0)),
        out_shape=jax.ShapeDtypeStruct((H_PAD, dout), jnp.float32),
    )(*args)


def _tc_combine(m, parts, dis, wb, ln_g, ln_b, do_ln, split_u):
    dout = m.shape[1]
    nparts = len(parts)

    def body(*refs):
        i = 0
        m_ref = refs[i]; i += 1
        p_refs = refs[i:i + nparts]; i += nparts
        dis_ref = refs[i]; i += 1
        wb_refs = refs[i:i + (nparts if wb is not None else 0)]
        i += len(wb_refs)
        if do_ln:
            lg_ref = refs[i]; i += 1
            lb_ref = refs[i]; i += 1
        h_ref = refs[i]; i += 1
        u_refs = refs[i:]

        dis_v = dis_ref[...]
        p = m_ref[...]
        for k in range(nparts):
            g = -dis_v * (p_refs[k][0] + p_refs[k][1])
            if wb is not None:
                p = p + jnp.dot(g, wb_refs[k][...],
                                preferred_element_type=jnp.float32,
                                precision=lax.Precision.HIGHEST)
            else:
                p = p + g
        if do_ln:
            mu = jnp.mean(p, axis=-1, keepdims=True)
            d = p - mu
            var = jnp.mean(d * d, axis=-1, keepdims=True)
            p = d * lax.rsqrt(var + _LN_EPS) * lg_ref[...] + lb_ref[...]
        pos = p > 0
        h = jnp.where(pos, p, jnp.exp(jnp.where(pos, 0.0, p)) - 1.0)
        h_ref[...] = h
        if split_u:
            u = dis_v * h
            u_refs[0][...] = u[:, :128]
            u_refs[1][...] = u[:, 128:]

    in_specs = [pl.BlockSpec((BLK, dout), lambda i: (i, 0))]
    args = [m]
    for pt in parts:
        in_specs.append(pl.BlockSpec((2, BLK, 128), lambda i: (0, i, 0)))
        args.append(pt)
    in_specs.append(pl.BlockSpec((BLK, 1), lambda i: (i, 0)))
    args.append(dis)
    if wb is not None:
        for k in range(nparts):
            in_specs.append(pl.BlockSpec((128, dout), lambda i: (0, 0)))
            args.append(wb[k * 128:(k + 1) * 128])
    if do_ln:
        in_specs.append(pl.BlockSpec((1, dout), lambda i: (0, 0)))
        args.append(ln_g.reshape(1, dout))
        in_specs.append(pl.BlockSpec((1, dout), lambda i: (0, 0)))
        args.append(ln_b.reshape(1, dout))

    out_specs = [pl.BlockSpec((BLK, dout), lambda i: (i, 0))]
    out_shape = [jax.ShapeDtypeStruct((H_PAD, dout), jnp.float32)]
    if split_u:
        for _ in range(2):
            out_specs.append(pl.BlockSpec((BLK, 128), lambda i: (i, 0)))
            out_shape.append(jax.ShapeDtypeStruct((H_PAD, 128), jnp.float32))

    out = pl.pallas_call(
        body,
        grid=(H_PAD // BLK,),
        in_specs=in_specs,
        out_specs=out_specs,
        out_shape=out_shape,
    )(*args)
    return out if split_u else (out[0],)


def kernel(x, edge_index, params):
    x = x.astype(jnp.float32)
    src = edge_index[0].astype(jnp.int32)
    dst = edge_index[1].astype(jnp.int32)
    e = src.shape[0]
    cpt = -(-e // (NUM_TILES * CHUNK))
    e_pad = NUM_TILES * CHUNK * cpt
    pad = e_pad - e
    pad_idx = (jnp.arange(pad, dtype=jnp.int32) % (H_PAD - N_NODES)) + N_NODES
    srcs = jnp.concatenate([src, pad_idx]).reshape(NUM_TILES, cpt, CHUNK)
    dsts = jnp.concatenate([dst, pad_idx]).reshape(NUM_TILES, cpt, CHUNK)
    x_pad = jnp.pad(x, ((0, H_PAD - x.shape[0]), (0, 0)))

    zeros128 = jnp.zeros((CHUNK, 128), jnp.float32)
    ones128 = jnp.ones((CHUNK, 128), jnp.float32)

    deg_p = _sc_degree(srcs, ones128, zeros128)
    dis, u0 = _tc_prep(deg_p, x_pad)

    layers = params["layers"]
    lns = params["ln"]
    num_layers = len(layers)

    h = x_pad
    u_halves = [u0]
    for li in range(num_layers):
        p0, p1 = layers[li][0], layers[li][1]
        s0 = jax.nn.sigmoid(p0["sw"])[0]
        s1 = jax.nn.sigmoid(p1["sw"])[0]
        wa = (p0["Ws"][0] * s0 + p1["Ws"][0] * s1) * 0.5
        wb = p1["Ws"][1] * (s1 * 0.5)
        bb = (p0["b"] * s0 + p1["b"] * s1) * 0.5
        last = li == num_layers - 1

        m = _tc_matmul(h, wa, b=bb)
        if last:
            u3 = _tc_matmul(h, wb, dis=dis)
            parts = [_sc_scatter(u3, srcs, dsts, zeros128)]
            (h,) = _tc_combine(m, parts, dis, None, None, None,
                               do_ln=False, split_u=False)
        else:
            parts = [_sc_scatter(u, srcs, dsts, zeros128) for u in u_halves]
            res = _tc_combine(m, parts, dis, wb, lns[li]["g"], lns[li]["b"],
                              do_ln=True, split_u=(li < num_layers - 2))
            if li < num_layers - 2:
                h, ua, ub = res
                u_halves = [ua, ub]
            else:
                (h,) = res
    return h[:N_NODES]


# speedup vs baseline: 2.9114x; 1.2487x over previous
import functools

import jax
import jax.numpy as jnp
from jax import lax
from jax.experimental import pallas as pl
from jax.experimental.pallas import tpu as pltpu
from jax.experimental.pallas import tpu_sc as plsc

N_NODES = 10000
NUM_CORES = 2
NUM_SUBCORES = 16
NUM_TILES = NUM_CORES * NUM_SUBCORES
CHUNK = 128
ROWS_PER_TILE = 640
H_PAD = ROWS_PER_TILE * NUM_SUBCORES
BLK = 1024
_LN_EPS = 1e-5


NBUF = 2
IDXW = 16


def _sc_degree(srcs, ones128, zeros128):
    cpt = srcs.shape[1]
    mesh = plsc.VectorSubcoreMesh(core_axis_name="c", subcore_axis_name="s")

    @functools.partial(
        pl.kernel,
        out_type=jax.ShapeDtypeStruct((NUM_CORES, H_PAD, 128), jnp.float32),
        mesh=mesh,
        scratch_types=[
            pltpu.VMEM((CHUNK,), jnp.int32),
            pltpu.VMEM((CHUNK, 128), jnp.float32),
            pltpu.VMEM_SHARED((H_PAD, 128), jnp.float32),
        ],
    )
    def deg_kernel(src_hbm, ones_hbm, z_hbm, out_hbm, sbuf, ones_v, acc):
        cid = lax.axis_index("c")
        sid = lax.axis_index("s")
        wid = sid * NUM_CORES + cid
        pltpu.sync_copy(ones_hbm, ones_v)
        for k in range(ROWS_PER_TILE // CHUNK):
            pltpu.sync_copy(
                z_hbm, acc.at[pl.ds(sid * ROWS_PER_TILE + k * CHUNK, CHUNK)])
        plsc.subcore_barrier()

        @pl.loop(0, cpt)
        def _(j):
            pltpu.sync_copy(src_hbm.at[wid, j], sbuf)
            pltpu.sync_copy(ones_v, acc.at[sbuf], add=True)

        plsc.subcore_barrier()
        pltpu.sync_copy(
            acc.at[pl.ds(sid * ROWS_PER_TILE, ROWS_PER_TILE)],
            out_hbm.at[cid, pl.ds(sid * ROWS_PER_TILE, ROWS_PER_TILE)])

    return deg_kernel(srcs, ones128, zeros128)


def _sc_scatter(u, srcs, dsts, zeros128):
    cpt = srcs.shape[1]
    mesh = plsc.VectorSubcoreMesh(core_axis_name="c", subcore_axis_name="s")

    scratch = ([pltpu.VMEM((CHUNK,), jnp.int32)] * 4
               + [pltpu.VMEM((CHUNK, 128), jnp.float32)] * 2
               + [pltpu.SemaphoreType.DMA] * 2
               + [pltpu.VMEM_SHARED((H_PAD, 128), jnp.float32)])

    @functools.partial(
        pl.kernel,
        out_type=jax.ShapeDtypeStruct((NUM_CORES, H_PAD, 128), jnp.float32),
        mesh=mesh,
        scratch_types=scratch,
    )
    def scat_kernel(u_hbm, src_hbm, dst_hbm, z_hbm, out_hbm,
                    sbuf0, dbuf0, sbuf1, dbuf1, rows0, rows1,
                    gsem0, gsem1, acc):
        cid = lax.axis_index("c")
        sid = lax.axis_index("s")
        wid = sid * NUM_CORES + cid
        pltpu.sync_copy(z_hbm, rows0)
        for k in range(ROWS_PER_TILE // CHUNK):
            pltpu.sync_copy(
                rows0, acc.at[pl.ds(sid * ROWS_PER_TILE + k * CHUNK, CHUNK)])
        plsc.subcore_barrier()

        @pl.loop(0, cpt, step=2)
        def _(j):
            pltpu.sync_copy(src_hbm.at[wid, j], sbuf0)
            pltpu.sync_copy(dst_hbm.at[wid, j], dbuf0)
            g0 = pltpu.async_copy(u_hbm.at[sbuf0], rows0, gsem0)
            pltpu.sync_copy(src_hbm.at[wid, j + 1], sbuf1)
            pltpu.sync_copy(dst_hbm.at[wid, j + 1], dbuf1)
            g0.wait()
            g1 = pltpu.async_copy(u_hbm.at[sbuf1], rows1, gsem1)
            pltpu.sync_copy(rows0, acc.at[dbuf0], add=True)
            g1.wait()
            pltpu.sync_copy(rows1, acc.at[dbuf1], add=True)

        plsc.subcore_barrier()
        pltpu.sync_copy(
            acc.at[pl.ds(sid * ROWS_PER_TILE, ROWS_PER_TILE)],
            out_hbm.at[cid, pl.ds(sid * ROWS_PER_TILE, ROWS_PER_TILE)])

    return scat_kernel(u, srcs, dsts, zeros128)


def _tc_prep(deg_p, x_pad):
    def body(dp_ref, x_ref, dis_ref, u_ref):
        deg = dp_ref[0, :, 0:1] + dp_ref[1, :, 0:1]
        dis = jnp.where(deg > 0, lax.rsqrt(deg), 0.0)
        dis_ref[...] = dis
        u_ref[...] = dis * x_ref[...]

    return pl.pallas_call(
        body,
        grid=(H_PAD // BLK,),
        in_specs=[pl.BlockSpec((2, BLK, 128), lambda i: (0, i, 0)),
                  pl.BlockSpec((BLK, 128), lambda i: (i, 0))],
        out_specs=[pl.BlockSpec((BLK, 1), lambda i: (i, 0)),
                   pl.BlockSpec((BLK, 128), lambda i: (i, 0))],
        out_shape=[jax.ShapeDtypeStruct((H_PAD, 1), jnp.float32),
                   jax.ShapeDtypeStruct((H_PAD, 128), jnp.float32)],
    )(deg_p, x_pad)


def _tc_matmul(h, w, b=None, dis=None):
    din, dout = w.shape
    n_in = 2 + (b is not None) + (dis is not None)

    def body(*refs):
        h_ref, w_ref = refs[0], refs[1]
        rest = list(refs[2:-1])
        o_ref = refs[-1]
        m = jnp.dot(h_ref[...], w_ref[...], preferred_element_type=jnp.float32,
                    precision=lax.Precision.HIGHEST)
        if b is not None:
            m = m + rest.pop(0)[...]
        if dis is not None:
            m = rest.pop(0)[...] * m
        o_ref[...] = m

    in_specs = [pl.BlockSpec((BLK, din), lambda i: (i, 0)),
                pl.BlockSpec((din, dout), lambda i: (0, 0))]
    args = [h, w]
    if b is not None:
        in_specs.append(pl.BlockSpec((1, dout), lambda i: (0, 0)))
        args.append(b.reshape(1, dout))
    if dis is not None:
        in_specs.append(pl.BlockSpec((BLK, 1), lambda i: (i, 0)))
        args.append(dis)
    del n_in
    return pl.pallas_call(
        body,
        grid=(H_PAD // BLK,),
        in_specs=in_specs,
        out_specs=pl.BlockSpec((BLK, dout), lambda i: (i, 0)),
        out_shape=jax.ShapeDtypeStruct((H_PAD, dout), jnp.float32),
    )(*args)


def _tc_combine(m, parts, dis, wb, ln_g, ln_b, do_ln, split_u):
    dout = m.shape[1]
    nparts = len(parts)

    def body(*refs):
        i = 0
        m_ref = refs[i]; i += 1
        p_refs = refs[i:i + nparts]; i += nparts
        dis_ref = refs[i]; i += 1
        wb_refs = refs[i:i + (nparts if wb is not None else 0)]
        i += len(wb_refs)
        if do_ln:
            lg_ref = refs[i]; i += 1
            lb_ref = refs[i]; i += 1
        h_ref = refs[i]; i += 1
        u_refs = refs[i:]

        dis_v = dis_ref[...]
        p = m_ref[...]
        for k in range(nparts):
            g = -dis_v * (p_refs[k][0] + p_refs[k][1])
            if wb is not None:
                p = p + jnp.dot(g, wb_refs[k][...],
                                preferred_element_type=jnp.float32,
                                precision=lax.Precision.HIGHEST)
            else:
                p = p + g
        if do_ln:
            mu = jnp.mean(p, axis=-1, keepdims=True)
            d = p - mu
            var = jnp.mean(d * d, axis=-1, keepdims=True)
            p = d * lax.rsqrt(var + _LN_EPS) * lg_ref[...] + lb_ref[...]
        pos = p > 0
        h = jnp.where(pos, p, jnp.exp(jnp.where(pos, 0.0, p)) - 1.0)
        h_ref[...] = h
        if split_u:
            u = dis_v * h
            u_refs[0][...] = u[:, :128]
            u_refs[1][...] = u[:, 128:]

    in_specs = [pl.BlockSpec((BLK, dout), lambda i: (i, 0))]
    args = [m]
    for pt in parts:
        in_specs.append(pl.BlockSpec((2, BLK, 128), lambda i: (0, i, 0)))
        args.append(pt)
    in_specs.append(pl.BlockSpec((BLK, 1), lambda i: (i, 0)))
    args.append(dis)
    if wb is not None:
        for k in range(nparts):
            in_specs.append(pl.BlockSpec((128, dout), lambda i: (0, 0)))
            args.append(wb[k * 128:(k + 1) * 128])
    if do_ln:
        in_specs.append(pl.BlockSpec((1, dout), lambda i: (0, 0)))
        args.append(ln_g.reshape(1, dout))
        in_specs.append(pl.BlockSpec((1, dout), lambda i: (0, 0)))
        args.append(ln_b.reshape(1, dout))

    out_specs = [pl.BlockSpec((BLK, dout), lambda i: (i, 0))]
    out_shape = [jax.ShapeDtypeStruct((H_PAD, dout), jnp.float32)]
    if split_u:
        for _ in range(2):
            out_specs.append(pl.BlockSpec((BLK, 128), lambda i: (i, 0)))
            out_shape.append(jax.ShapeDtypeStruct((H_PAD, 128), jnp.float32))

    out = pl.pallas_call(
        body,
        grid=(H_PAD // BLK,),
        in_specs=in_specs,
        out_specs=out_specs,
        out_shape=out_shape,
    )(*args)
    return out if split_u else (out[0],)


def kernel(x, edge_index, params):
    x = x.astype(jnp.float32)
    src = edge_index[0].astype(jnp.int32)
    dst = edge_index[1].astype(jnp.int32)
    e = src.shape[0]
    cpt = -(-e // (NUM_TILES * CHUNK))
    cpt = -(-cpt // 2) * 2
    e_pad = NUM_TILES * CHUNK * cpt
    pad = e_pad - e
    pad_idx = (jnp.arange(pad, dtype=jnp.int32) % (H_PAD - N_NODES)) + N_NODES
    srcs = jnp.concatenate([src, pad_idx]).reshape(NUM_TILES, cpt, CHUNK)
    dsts = jnp.concatenate([dst, pad_idx]).reshape(NUM_TILES, cpt, CHUNK)
    x_pad = jnp.pad(x, ((0, H_PAD - x.shape[0]), (0, 0)))

    zeros128 = jnp.zeros((CHUNK, 128), jnp.float32)
    ones128 = jnp.ones((CHUNK, 128), jnp.float32)

    deg_p = _sc_degree(srcs, ones128, zeros128)
    dis, u0 = _tc_prep(deg_p, x_pad)

    layers = params["layers"]
    lns = params["ln"]
    num_layers = len(layers)

    h = x_pad
    u_halves = [u0]
    for li in range(num_layers):
        p0, p1 = layers[li][0], layers[li][1]
        s0 = jax.nn.sigmoid(p0["sw"])[0]
        s1 = jax.nn.sigmoid(p1["sw"])[0]
        wa = (p0["Ws"][0] * s0 + p1["Ws"][0] * s1) * 0.5
        wb = p1["Ws"][1] * (s1 * 0.5)
        bb = (p0["b"] * s0 + p1["b"] * s1) * 0.5
        last = li == num_layers - 1

        m = _tc_matmul(h, wa, b=bb)
        if last:
            u3 = _tc_matmul(h, wb, dis=dis)
            parts = [_sc_scatter(u3, srcs, dsts, zeros128)]
            (h,) = _tc_combine(m, parts, dis, None, None, None,
                               do_ln=False, split_u=False)
        else:
            parts = [_sc_scatter(u, srcs, dsts, zeros128) for u in u_halves]
            res = _tc_combine(m, parts, dis, wb, lns[li]["g"], lns[li]["b"],
                              do_ln=True, split_u=(li < num_layers - 2))
            if li < num_layers - 2:
                h, ua, ub = res
                u_halves = [ua, ub]
            else:
                (h,) = res
    return h[:N_NODES]


# speedup vs baseline: 2.9545x; 1.0148x over previous
import functools

import jax
import jax.numpy as jnp
from jax import lax
from jax.experimental import pallas as pl
from jax.experimental.pallas import tpu as pltpu
from jax.experimental.pallas import tpu_sc as plsc

N_NODES = 10000
NUM_CORES = 2
NUM_SUBCORES = 16
NUM_TILES = NUM_CORES * NUM_SUBCORES
CHUNK = 128
ROWS_PER_TILE = 640
H_PAD = ROWS_PER_TILE * NUM_SUBCORES
BLK = 1024
_LN_EPS = 1e-5


NBUF = 2
IDXW = 16


def _sc_degree(srcs, ones128, zeros128):
    cpt = srcs.shape[1]
    mesh = plsc.VectorSubcoreMesh(core_axis_name="c", subcore_axis_name="s")

    @functools.partial(
        pl.kernel,
        out_type=jax.ShapeDtypeStruct((NUM_CORES, H_PAD, 128), jnp.float32),
        mesh=mesh,
        scratch_types=[
            pltpu.VMEM((CHUNK,), jnp.int32),
            pltpu.VMEM((CHUNK,), jnp.int32),
            pltpu.VMEM((CHUNK, 128), jnp.float32),
            pltpu.SemaphoreType.DMA,
            pltpu.SemaphoreType.DMA,
            pltpu.VMEM_SHARED((H_PAD, 128), jnp.float32),
        ],
    )
    def deg_kernel(src_hbm, ones_hbm, z_hbm, out_hbm,
                   sbuf0, sbuf1, ones_v, sem0, sem1, acc):
        cid = lax.axis_index("c")
        sid = lax.axis_index("s")
        wid = sid * NUM_CORES + cid
        pltpu.sync_copy(ones_hbm, ones_v)
        for k in range(ROWS_PER_TILE // CHUNK):
            pltpu.sync_copy(
                z_hbm, acc.at[pl.ds(sid * ROWS_PER_TILE + k * CHUNK, CHUNK)])
        plsc.subcore_barrier()

        @pl.loop(0, cpt, step=2)
        def _(j):
            pltpu.sync_copy(src_hbm.at[wid, j], sbuf0)
            s0 = pltpu.async_copy(ones_v, acc.at[sbuf0], sem0, add=True)
            pltpu.sync_copy(src_hbm.at[wid, j + 1], sbuf1)
            s1 = pltpu.async_copy(ones_v, acc.at[sbuf1], sem1, add=True)
            s0.wait()
            s1.wait()

        plsc.subcore_barrier()
        pltpu.sync_copy(
            acc.at[pl.ds(sid * ROWS_PER_TILE, ROWS_PER_TILE)],
            out_hbm.at[cid, pl.ds(sid * ROWS_PER_TILE, ROWS_PER_TILE)])

    return deg_kernel(srcs, ones128, zeros128)


def _sc_scatter(u, srcs, dsts, zeros128):
    cpt = srcs.shape[1]
    mesh = plsc.VectorSubcoreMesh(core_axis_name="c", subcore_axis_name="s")

    scratch = ([pltpu.VMEM((CHUNK,), jnp.int32)] * 4
               + [pltpu.VMEM((CHUNK, 128), jnp.float32)] * 2
               + [pltpu.SemaphoreType.DMA] * 2
               + [pltpu.VMEM_SHARED((H_PAD, 128), jnp.float32)])

    @functools.partial(
        pl.kernel,
        out_type=jax.ShapeDtypeStruct((NUM_CORES, H_PAD, 128), jnp.float32),
        mesh=mesh,
        scratch_types=scratch,
    )
    def scat_kernel(u_hbm, src_hbm, dst_hbm, z_hbm, out_hbm,
                    sbuf0, dbuf0, sbuf1, dbuf1, rows0, rows1,
                    gsem0, gsem1, acc):
        cid = lax.axis_index("c")
        sid = lax.axis_index("s")
        wid = sid * NUM_CORES + cid
        pltpu.sync_copy(z_hbm, rows0)
        for k in range(ROWS_PER_TILE // CHUNK):
            pltpu.sync_copy(
                rows0, acc.at[pl.ds(sid * ROWS_PER_TILE + k * CHUNK, CHUNK)])
        plsc.subcore_barrier()

        @pl.loop(0, cpt, step=2)
        def _(j):
            pltpu.sync_copy(src_hbm.at[wid, j], sbuf0)
            pltpu.sync_copy(dst_hbm.at[wid, j], dbuf0)
            g0 = pltpu.async_copy(u_hbm.at[sbuf0], rows0, gsem0)
            pltpu.sync_copy(src_hbm.at[wid, j + 1], sbuf1)
            pltpu.sync_copy(dst_hbm.at[wid, j + 1], dbuf1)
            g0.wait()
            g1 = pltpu.async_copy(u_hbm.at[sbuf1], rows1, gsem1)
            pltpu.sync_copy(rows0, acc.at[dbuf0], add=True)
            g1.wait()
            pltpu.sync_copy(rows1, acc.at[dbuf1], add=True)

        plsc.subcore_barrier()
        pltpu.sync_copy(
            acc.at[pl.ds(sid * ROWS_PER_TILE, ROWS_PER_TILE)],
            out_hbm.at[cid, pl.ds(sid * ROWS_PER_TILE, ROWS_PER_TILE)])

    return scat_kernel(u, srcs, dsts, zeros128)


def _tc_prep(deg_p, x_pad):
    def body(dp_ref, x_ref, dis_ref, u_ref):
        deg = dp_ref[0, :, 0:1] + dp_ref[1, :, 0:1]
        dis = jnp.where(deg > 0, lax.rsqrt(deg), 0.0)
        dis_ref[...] = dis
        u_ref[...] = dis * x_ref[...]

    return pl.pallas_call(
        body,
        grid=(H_PAD // BLK,),
        in_specs=[pl.BlockSpec((2, BLK, 128), lambda i: (0, i, 0)),
                  pl.BlockSpec((BLK, 128), lambda i: (i, 0))],
        out_specs=[pl.BlockSpec((BLK, 1), lambda i: (i, 0)),
                   pl.BlockSpec((BLK, 128), lambda i: (i, 0))],
        out_shape=[jax.ShapeDtypeStruct((H_PAD, 1), jnp.float32),
                   jax.ShapeDtypeStruct((H_PAD, 128), jnp.float32)],
    )(deg_p, x_pad)


def _tc_matmul(h, w, b=None, dis=None):
    din, dout = w.shape
    n_in = 2 + (b is not None) + (dis is not None)

    def body(*refs):
        h_ref, w_ref = refs[0], refs[1]
        rest = list(refs[2:-1])
        o_ref = refs[-1]
        m = jnp.dot(h_ref[...], w_ref[...], preferred_element_type=jnp.float32,
                    precision=lax.Precision.HIGHEST)
        if b is not None:
            m = m + rest.pop(0)[...]
        if dis is not None:
            m = rest.pop(0)[...] * m
        o_ref[...] = m

    in_specs = [pl.BlockSpec((BLK, din), lambda i: (i, 0)),
                pl.BlockSpec((din, dout), lambda i: (0, 0))]
    args = [h, w]
    if b is not None:
        in_specs.append(pl.BlockSpec((1, dout), lambda i: (0, 0)))
        args.append(b.reshape(1, dout))
    if dis is not None:
        in_specs.append(pl.BlockSpec((BLK, 1), lambda i: (i, 0)))
        args.append(dis)
    del n_in
    return pl.pallas_call(
        body,
        grid=(H_PAD // BLK,),
        in_specs=in_specs,
        out_specs=pl.BlockSpec((BLK, dout), lambda i: (i, 0)),
        out_shape=jax.ShapeDtypeStruct((H_PAD, dout), jnp.float32),
    )(*args)


def _tc_combine(m, parts, dis, wb, ln_g, ln_b, do_ln, split_u):
    dout = m.shape[1]
    nparts = len(parts)

    def body(*refs):
        i = 0
        m_ref = refs[i]; i += 1
        p_refs = refs[i:i + nparts]; i += nparts
        dis_ref = refs[i]; i += 1
        wb_refs = refs[i:i + (nparts if wb is not None else 0)]
        i += len(wb_refs)
        if do_ln:
            lg_ref = refs[i]; i += 1
            lb_ref = refs[i]; i += 1
        h_ref = refs[i]; i += 1
        u_refs = refs[i:]

        dis_v = dis_ref[...]
        p = m_ref[...]
        for k in range(nparts):
            g = -dis_v * (p_refs[k][0] + p_refs[k][1])
            if wb is not None:
                p = p + jnp.dot(g, wb_refs[k][...],
                                preferred_element_type=jnp.float32,
                                precision=lax.Precision.HIGHEST)
            else:
                p = p + g
        if do_ln:
            mu = jnp.mean(p, axis=-1, keepdims=True)
            d = p - mu
            var = jnp.mean(d * d, axis=-1, keepdims=True)
            p = d * lax.rsqrt(var + _LN_EPS) * lg_ref[...] + lb_ref[...]
        pos = p > 0
        h = jnp.where(pos, p, jnp.exp(jnp.where(pos, 0.0, p)) - 1.0)
        h_ref[...] = h
        if split_u:
            u = dis_v * h
            u_refs[0][...] = u[:, :128]
            u_refs[1][...] = u[:, 128:]

    in_specs = [pl.BlockSpec((BLK, dout), lambda i: (i, 0))]
    args = [m]
    for pt in parts:
        in_specs.append(pl.BlockSpec((2, BLK, 128), lambda i: (0, i, 0)))
        args.append(pt)
    in_specs.append(pl.BlockSpec((BLK, 1), lambda i: (i, 0)))
    args.append(dis)
    if wb is not None:
        for k in range(nparts):
            in_specs.append(pl.BlockSpec((128, dout), lambda i: (0, 0)))
            args.append(wb[k * 128:(k + 1) * 128])
    if do_ln:
        in_specs.append(pl.BlockSpec((1, dout), lambda i: (0, 0)))
        args.append(ln_g.reshape(1, dout))
        in_specs.append(pl.BlockSpec((1, dout), lambda i: (0, 0)))
        args.append(ln_b.reshape(1, dout))

    out_specs = [pl.BlockSpec((BLK, dout), lambda i: (i, 0))]
    out_shape = [jax.ShapeDtypeStruct((H_PAD, dout), jnp.float32)]
    if split_u:
        for _ in range(2):
            out_specs.append(pl.BlockSpec((BLK, 128), lambda i: (i, 0)))
            out_shape.append(jax.ShapeDtypeStruct((H_PAD, 128), jnp.float32))

    out = pl.pallas_call(
        body,
        grid=(H_PAD // BLK,),
        in_specs=in_specs,
        out_specs=out_specs,
        out_shape=out_shape,
    )(*args)
    return out if split_u else (out[0],)


def kernel(x, edge_index, params):
    x = x.astype(jnp.float32)
    src = edge_index[0].astype(jnp.int32)
    dst = edge_index[1].astype(jnp.int32)
    e = src.shape[0]
    cpt = -(-e // (NUM_TILES * CHUNK))
    cpt = -(-cpt // 2) * 2
    e_pad = NUM_TILES * CHUNK * cpt
    pad = e_pad - e
    pad_idx = (jnp.arange(pad, dtype=jnp.int32) % (H_PAD - N_NODES)) + N_NODES
    srcs = jnp.concatenate([src, pad_idx]).reshape(NUM_TILES, cpt, CHUNK)
    dsts = jnp.concatenate([dst, pad_idx]).reshape(NUM_TILES, cpt, CHUNK)
    x_pad = jnp.pad(x, ((0, H_PAD - x.shape[0]), (0, 0)))

    zeros128 = jnp.zeros((CHUNK, 128), jnp.float32)
    ones128 = jnp.ones((CHUNK, 128), jnp.float32)

    deg_p = _sc_degree(srcs, ones128, zeros128)
    dis, u0 = _tc_prep(deg_p, x_pad)

    layers = params["layers"]
    lns = params["ln"]
    num_layers = len(layers)

    h = x_pad
    u_halves = [u0]
    for li in range(num_layers):
        p0, p1 = layers[li][0], layers[li][1]
        s0 = jax.nn.sigmoid(p0["sw"])[0]
        s1 = jax.nn.sigmoid(p1["sw"])[0]
        wa = (p0["Ws"][0] * s0 + p1["Ws"][0] * s1) * 0.5
        wb = p1["Ws"][1] * (s1 * 0.5)
        bb = (p0["b"] * s0 + p1["b"] * s1) * 0.5
        last = li == num_layers - 1

        m = _tc_matmul(h, wa, b=bb)
        if last:
            u3 = _tc_matmul(h, wb, dis=dis)
            parts = [_sc_scatter(u3, srcs, dsts, zeros128)]
            (h,) = _tc_combine(m, parts, dis, None, None, None,
                               do_ln=False, split_u=False)
        else:
            parts = [_sc_scatter(u, srcs, dsts, zeros128) for u in u_halves]
            res = _tc_combine(m, parts, dis, wb, lns[li]["g"], lns[li]["b"],
                              do_ln=True, split_u=(li < num_layers - 2))
            if li < num_layers - 2:
                h, ua, ub = res
                u_halves = [ua, ub]
            else:
                (h,) = res
    return h[:N_NODES]
